# Initial kernel scaffold; baseline (speedup 1.0000x reference)
import jax
import jax.numpy as jnp
from jax import lax
from jax.experimental import pallas as pl
from jax.experimental.pallas import tpu as pltpu
from jax.experimental.pallas import tpu_sc as plsc

NC = 2
NS = 16
NW = NC * NS

_MESH = plsc.VectorSubcoreMesh(
    core_axis_name="c", subcore_axis_name="s", num_cores=NC, num_subcores=NS
)


def _deg_body(dst_hbm, zeros1, out_hbm, dst_v, ones_v, deg_sh, sem):
    c = lax.axis_index("c")
    s = lax.axis_index("s")
    wid = c * NS + s
    nch = dst_v.shape[0]

    @pl.when(s == 0)
    def _():
        pltpu.sync_copy(zeros1, deg_sh)

    for k in range(ones_v.shape[0] // 16):
        ones_v[pl.ds(16 * k, 16)] = jnp.ones((16,), jnp.float32)
    pltpu.sync_copy(dst_hbm.at[wid], dst_v)
    plsc.subcore_barrier()

    def step(j, carry):
        pltpu.sync_copy(ones_v, deg_sh.at[dst_v.at[j]], add=True)
        return carry

    lax.fori_loop(0, nch, step, 0)
    plsc.subcore_barrier()

    @pl.when(s == 0)
    def _():
        pltpu.sync_copy(deg_sh, out_hbm.at[c])


def _agg_body(hs_hbm, src_hbm, dst_hbm, zeros2, out_hbm,
              src_v, dst_v, rows_a, rows_b, sem_a, sem_b, acc_sh):
    c = lax.axis_index("c")
    s = lax.axis_index("s")
    wid = c * NS + s
    nch = src_v.shape[0]
    n = acc_sh.shape[0]
    rpt = n // NS

    pltpu.sync_copy(zeros2.at[pl.ds(s * rpt, rpt)], acc_sh.at[pl.ds(s * rpt, rpt)])
    pltpu.sync_copy(src_hbm.at[wid], src_v)
    pltpu.sync_copy(dst_hbm.at[wid], dst_v)
    pltpu.async_copy(hs_hbm.at[src_v.at[0]], rows_a, sem_a)
    plsc.subcore_barrier()

    def pair(g, carry):
        j0 = 2 * g
        j1 = j0 + 1
        j2 = j0 + 2

        @pl.when(j1 < nch)
        def _():
            pltpu.async_copy(hs_hbm.at[src_v.at[j1]], rows_b, sem_b)

        pltpu.make_async_copy(hs_hbm.at[src_v.at[j0]], rows_a, sem_a).wait()
        pltpu.sync_copy(rows_a, acc_sh.at[dst_v.at[j0]], add=True)

        @pl.when(j2 < nch)
        def _():
            pltpu.async_copy(hs_hbm.at[src_v.at[j2]], rows_a, sem_a)

        @pl.when(j1 < nch)
        def _():
            pltpu.make_async_copy(hs_hbm.at[src_v.at[j1]], rows_b, sem_b).wait()
            pltpu.sync_copy(rows_b, acc_sh.at[dst_v.at[j1]], add=True)

        return carry

    lax.fori_loop(0, (nch + 1) // 2, pair, 0)
    plsc.subcore_barrier()
    pltpu.sync_copy(acc_sh.at[pl.ds(s * rpt, rpt)],
                    out_hbm.at[c, pl.ds(s * rpt, rpt)])


def _make_sc_kernels(n, d, nch, ch):
    deg_k = pl.kernel(
        _deg_body,
        out_type=jax.ShapeDtypeStruct((NC, n), jnp.float32),
        mesh=_MESH,
        scratch_types=[
            pltpu.VMEM((nch, ch), jnp.int32),
            pltpu.VMEM((ch,), jnp.float32),
            pltpu.VMEM_SHARED((n,), jnp.float32),
            pltpu.SemaphoreType.DMA,
        ],
    )
    agg_k = pl.kernel(
        _agg_body,
        out_type=jax.ShapeDtypeStruct((NC, n, d), jnp.float32),
        mesh=_MESH,
        scratch_types=[
            pltpu.VMEM((nch, ch), jnp.int32),
            pltpu.VMEM((nch, ch), jnp.int32),
            pltpu.VMEM((ch, d), jnp.float32),
            pltpu.VMEM((ch, d), jnp.float32),
            pltpu.SemaphoreType.DMA,
            pltpu.SemaphoreType.DMA,
            pltpu.VMEM_SHARED((n, d), jnp.float32),
        ],
    )
    return deg_k, agg_k


def _tc1_body(x_ref, w1_ref, degp_ref, h1s_ref, dis_ref):
    deg = degp_ref[0] + degp_ref[1] + 1.0
    dis = lax.rsqrt(deg)
    h = jnp.dot(x_ref[...], w1_ref[...], preferred_element_type=jnp.float32)
    h1s_ref[...] = h * dis
    dis_ref[...] = dis


def _tc2_body(agg_ref, h1s_ref, dis_ref, b1_ref, w2_ref, h2s_ref):
    pre = (agg_ref[0] + agg_ref[1] + h1s_ref[...]) * dis_ref[...] + b1_ref[...]
    h1 = jnp.maximum(pre, 0.0)
    h2 = jnp.dot(h1, w2_ref[...], preferred_element_type=jnp.float32)
    h2s_ref[...] = h2 * dis_ref[...]


def _tc3_body(agg_ref, h2s_ref, dis_ref, b2_ref, out_ref):
    out_ref[...] = (agg_ref[0] + agg_ref[1] + h2s_ref[...]) * dis_ref[...] \
        + b2_ref[...]


def kernel(x, edge_index, W1, b1, W2, b2):
    n, _ = x.shape
    d_hid = W1.shape[1]
    d_out = W2.shape[1]
    e = edge_index.shape[1]

    per_w = e // NW
    ch = 80
    while per_w % ch:
        ch -= 8
    nch = per_w // ch

    ei = edge_index.astype(jnp.int32)
    src = ei[0].reshape(NW, nch, ch)
    dst = ei[1].reshape(NW, nch, ch)
    zeros1 = jnp.zeros((n,), jnp.float32)
    zeros2 = jnp.zeros((n, d_hid), jnp.float32)

    deg_k, agg_k = _make_sc_kernels(n, d_hid, nch, ch)

    degp = deg_k(dst, zeros1)
    degp3 = degp.reshape(NC, n, 1)

    tc1 = pl.pallas_call(
        _tc1_body,
        out_shape=(
            jax.ShapeDtypeStruct((n, d_hid), jnp.float32),
            jax.ShapeDtypeStruct((n, 1), jnp.float32),
        ),
    )
    h1s, dis = tc1(x, W1, degp3)

    agg1 = agg_k(h1s, src, dst, zeros2)

    tc2 = pl.pallas_call(
        _tc2_body,
        out_shape=jax.ShapeDtypeStruct((n, d_hid), jnp.float32),
    )
    h2s = tc2(agg1, h1s, dis, b1.reshape(1, d_hid), W2)

    agg2 = agg_k(h2s, src, dst, zeros2)

    tc3 = pl.pallas_call(
        _tc3_body,
        out_shape=jax.ShapeDtypeStruct((n, d_out), jnp.float32),
    )
    out = tc3(agg2, h2s, dis, b2.reshape(1, d_out))
    return out


# speedup vs baseline: 21.1207x; 21.1207x over previous
import jax
import jax.numpy as jnp
from jax import lax
from jax.experimental import pallas as pl
from jax.experimental.pallas import tpu as pltpu
from jax.experimental.pallas import tpu_sc as plsc

NC = 2
NS = 16
NW = NC * NS
DH = 64

_MESH = plsc.VectorSubcoreMesh(
    core_axis_name="c", subcore_axis_name="s", num_cores=NC, num_subcores=NS
)


def _tile_rows(n):
    base = ((n + NS - 1) // NS + 7) // 8 * 8
    last = n - base * (NS - 1)
    assert 0 < last <= base and last % 8 == 0
    return base, last


def _deg_body(dst_hbm, zeros1, out_hbm, dst_v, ones_v, deg_sh, sem):
    c = lax.axis_index("c")
    s = lax.axis_index("s")
    wid = c * NS + s
    nch = dst_v.shape[0]
    n1 = deg_sh.shape[0]

    @pl.when(s == 0)
    def _():
        pltpu.sync_copy(zeros1, deg_sh)

    for k in range(ones_v.shape[0] // 16):
        ones_v[pl.ds(16 * k, 16)] = jnp.ones((16,), jnp.float32)
    pltpu.sync_copy(dst_hbm.at[wid], dst_v)
    plsc.subcore_barrier()

    def step(j, carry):
        pltpu.sync_copy(ones_v, deg_sh.at[dst_v.at[j]], add=True)
        return carry

    lax.fori_loop(0, nch, step, 0)
    plsc.subcore_barrier()

    @pl.when(s == 0)
    def _():
        pltpu.sync_copy(deg_sh, out_hbm.at[pl.ds(pl.multiple_of(c * n1, 128), n1)])


def _agg_body(hs0, hs1, src_hbm, dst_hbm, zeros2, out_hbm,
              src_v, dst_v, rows_a, rows_b, sem_a, sem_b, acc_sh):
    c = lax.axis_index("c")
    s = lax.axis_index("s")
    wid = c * NS + s
    nch = src_v.shape[0]
    n = acc_sh.shape[0]
    base, last = _tile_rows(n)
    off = pl.multiple_of(s * base, 8)
    lo_last = base * (NS - 1)

    pltpu.sync_copy(src_hbm.at[wid], src_v)
    pltpu.sync_copy(dst_hbm.at[wid], dst_v)

    for half, hs in enumerate((hs0, hs1)):
        @pl.when(s < NS - 1)
        def _():
            pltpu.sync_copy(zeros2.at[pl.ds(off, base)],
                            acc_sh.at[pl.ds(off, base)])

        @pl.when(s == NS - 1)
        def _():
            pltpu.sync_copy(zeros2.at[pl.ds(lo_last, last)],
                            acc_sh.at[pl.ds(lo_last, last)])

        pltpu.async_copy(hs.at[src_v.at[0]], rows_a, sem_a)
        plsc.subcore_barrier()

        def pair(g, carry, hs=hs):
            j0 = 2 * g
            j1 = j0 + 1
            j2 = j0 + 2

            @pl.when(j1 < nch)
            def _():
                pltpu.async_copy(hs.at[src_v.at[j1]], rows_b, sem_b)

            pltpu.make_async_copy(hs.at[src_v.at[j0]], rows_a, sem_a).wait()
            pltpu.sync_copy(rows_a, acc_sh.at[dst_v.at[j0]], add=True)

            @pl.when(j2 < nch)
            def _():
                pltpu.async_copy(hs.at[src_v.at[j2]], rows_a, sem_a)

            @pl.when(j1 < nch)
            def _():
                pltpu.make_async_copy(hs.at[src_v.at[j1]], rows_b, sem_b).wait()
                pltpu.sync_copy(rows_b, acc_sh.at[dst_v.at[j1]], add=True)

            return carry

        lax.fori_loop(0, (nch + 1) // 2, pair, 0)
        plsc.subcore_barrier()

        @pl.when(s < NS - 1)
        def _():
            pltpu.sync_copy(acc_sh.at[pl.ds(off, base)],
                            out_hbm.at[half, c, pl.ds(off, base)])

        @pl.when(s == NS - 1)
        def _():
            pltpu.sync_copy(acc_sh.at[pl.ds(lo_last, last)],
                            out_hbm.at[half, c, pl.ds(lo_last, last)])


def _make_sc_kernels(n, nch, ch):
    n1 = (n + 127) // 128 * 128
    deg_k = pl.kernel(
        _deg_body,
        out_type=jax.ShapeDtypeStruct((NC * n1,), jnp.float32),
        mesh=_MESH,
        scratch_types=[
            pltpu.VMEM((nch, ch), jnp.int32),
            pltpu.VMEM((ch,), jnp.float32),
            pltpu.VMEM_SHARED((n1,), jnp.float32),
            pltpu.SemaphoreType.DMA,
        ],
    )
    agg_k = pl.kernel(
        _agg_body,
        out_type=jax.ShapeDtypeStruct((2, NC, n, DH), jnp.float32),
        mesh=_MESH,
        compiler_params=pltpu.CompilerParams(use_tc_tiling_on_sc=False),
        scratch_types=[
            pltpu.VMEM((nch, ch), jnp.int32),
            pltpu.VMEM((nch, ch), jnp.int32),
            pltpu.VMEM((ch, DH), jnp.float32),
            pltpu.VMEM((ch, DH), jnp.float32),
            pltpu.SemaphoreType.DMA,
            pltpu.SemaphoreType.DMA,
            pltpu.VMEM_SHARED((n, DH), jnp.float32),
        ],
    )
    return deg_k, agg_k


def _tc1_body(x_ref, w1_ref, degp_ref, h1s0_ref, h1s1_ref, dis_ref):
    deg = degp_ref[0] + degp_ref[1] + 1.0
    dis = lax.rsqrt(deg)
    h = jnp.dot(x_ref[...], w1_ref[...], preferred_element_type=jnp.float32)
    hs = h * dis
    h1s0_ref[...] = hs[:, :DH]
    h1s1_ref[...] = hs[:, DH:]
    dis_ref[...] = dis


def _tc2_body(agg_ref, h1s0_ref, h1s1_ref, dis_ref, b1_ref, w2_ref,
              h2s0_ref, h2s1_ref):
    dis = dis_ref[...]
    b1 = b1_ref[...]
    pre0 = (agg_ref[0, 0] + agg_ref[0, 1] + h1s0_ref[...]) * dis + b1[:, :DH]
    pre1 = (agg_ref[1, 0] + agg_ref[1, 1] + h1s1_ref[...]) * dis + b1[:, DH:]
    h1 = jnp.concatenate([jnp.maximum(pre0, 0.0), jnp.maximum(pre1, 0.0)],
                         axis=1)
    h2 = jnp.dot(h1, w2_ref[...], preferred_element_type=jnp.float32)
    hs = h2 * dis
    h2s0_ref[...] = hs[:, :DH]
    h2s1_ref[...] = hs[:, DH:]


def _tc3_body(agg_ref, h2s0_ref, h2s1_ref, dis_ref, b2_ref, out_ref):
    dis = dis_ref[...]
    b2 = b2_ref[...]
    o0 = (agg_ref[0, 0] + agg_ref[0, 1] + h2s0_ref[...]) * dis + b2[:, :DH]
    o1 = (agg_ref[1, 0] + agg_ref[1, 1] + h2s1_ref[...]) * dis + b2[:, DH:]
    out_ref[...] = jnp.concatenate([o0, o1], axis=1)


def kernel(x, edge_index, W1, b1, W2, b2):
    n, _ = x.shape
    d_hid = W1.shape[1]
    d_out = W2.shape[1]
    e = edge_index.shape[1]
    assert d_hid == 2 * DH and d_out == 2 * DH

    per_w = e // NW
    ch = 80
    while per_w % ch:
        ch -= 8
    nch = per_w // ch

    ei = edge_index.astype(jnp.int32)
    src = ei[0].reshape(NW, nch, ch)
    dst = ei[1].reshape(NW, nch, ch)
    n1 = (n + 127) // 128 * 128
    zeros1 = jnp.zeros((n1,), jnp.float32)
    zeros2 = jnp.zeros((n, DH), jnp.float32)

    deg_k, agg_k = _make_sc_kernels(n, nch, ch)

    degp = deg_k(dst, zeros1)
    degp3 = degp.reshape(NC, n1, 1)[:, :n, :]

    tc1 = pl.pallas_call(
        _tc1_body,
        out_shape=(
            jax.ShapeDtypeStruct((n, DH), jnp.float32),
            jax.ShapeDtypeStruct((n, DH), jnp.float32),
            jax.ShapeDtypeStruct((n, 1), jnp.float32),
        ),
    )
    h1s0, h1s1, dis = tc1(x, W1, degp3)

    agg1 = agg_k(h1s0, h1s1, src, dst, zeros2)

    tc2 = pl.pallas_call(
        _tc2_body,
        out_shape=(
            jax.ShapeDtypeStruct((n, DH), jnp.float32),
            jax.ShapeDtypeStruct((n, DH), jnp.float32),
        ),
    )
    h2s0, h2s1 = tc2(agg1, h1s0, h1s1, dis, b1.reshape(1, d_hid), W2)

    agg2 = agg_k(h2s0, h2s1, src, dst, zeros2)

    tc3 = pl.pallas_call(
        _tc3_body,
        out_shape=jax.ShapeDtypeStruct((n, d_out), jnp.float32),
    )
    out = tc3(agg2, h2s0, h2s1, dis, b2.reshape(1, d_out))
    return out


# speedup vs baseline: 26.3399x; 1.2471x over previous
import jax
import jax.numpy as jnp
from jax import lax
from jax.experimental import pallas as pl
from jax.experimental.pallas import tpu as pltpu
from jax.experimental.pallas import tpu_sc as plsc

NC = 2
NS = 16
NW = NC * NS
DH = 64

_MESH = plsc.VectorSubcoreMesh(
    core_axis_name="c", subcore_axis_name="s", num_cores=NC, num_subcores=NS
)


def _tile_rows(n):
    base = ((n + NS - 1) // NS + 7) // 8 * 8
    last = n - base * (NS - 1)
    assert 0 < last <= base and last % 8 == 0
    return base, last


def _deg_body(dst_hbm, zeros1, out_hbm, dst_v, ones_v, deg_sh, sem):
    c = lax.axis_index("c")
    s = lax.axis_index("s")
    wid = c * NS + s
    nch = dst_v.shape[0]
    n1 = deg_sh.shape[0]

    @pl.when(s == 0)
    def _():
        pltpu.sync_copy(zeros1, deg_sh)

    for k in range(ones_v.shape[0] // 16):
        ones_v[pl.ds(16 * k, 16)] = jnp.ones((16,), jnp.float32)
    pltpu.sync_copy(dst_hbm.at[wid], dst_v)
    plsc.subcore_barrier()

    def step(j, carry):
        pltpu.sync_copy(ones_v, deg_sh.at[dst_v.at[j]], add=True)
        return carry

    lax.fori_loop(0, nch, step, 0)
    plsc.subcore_barrier()

    @pl.when(s == 0)
    def _():
        pltpu.sync_copy(deg_sh, out_hbm.at[pl.ds(pl.multiple_of(c * n1, 128), n1)])


def _agg_body(hs0, hs1, src_hbm, dst_hbm, zeros2, out_hbm,
              src_v, dst_v, rows_0, rows_1, rows_2, rows_3,
              gsem, ssem, acc_sh):
    c = lax.axis_index("c")
    s = lax.axis_index("s")
    wid = c * NS + s
    nch = src_v.shape[0]
    assert nch >= 2
    n = acc_sh.shape[0]
    base, last = _tile_rows(n)
    off = pl.multiple_of(s * base, 8)
    lo_last = base * (NS - 1)
    rows = (rows_0, rows_1, rows_2, rows_3)

    pltpu.sync_copy(src_hbm.at[wid], src_v)
    pltpu.sync_copy(dst_hbm.at[wid], dst_v)

    for half, hs in enumerate((hs0, hs1)):
        @pl.when(s < NS - 1)
        def _():
            pltpu.sync_copy(zeros2.at[pl.ds(off, base)],
                            acc_sh.at[pl.ds(off, base)])

        @pl.when(s == NS - 1)
        def _():
            pltpu.sync_copy(zeros2.at[pl.ds(lo_last, last)],
                            acc_sh.at[pl.ds(lo_last, last)])

        pltpu.async_copy(hs.at[src_v.at[0]], rows[0], gsem.at[0])
        pltpu.async_copy(hs.at[src_v.at[1]], rows[1], gsem.at[1])
        plsc.subcore_barrier()

        def quad(g, carry, hs=hs):
            for u in range(4):
                t = 4 * g + u
                b_cur = u
                b_pre = (u + 2) % 4

                @pl.when((t >= 2) & (t < nch + 2))
                def _():
                    pltpu.make_async_copy(
                        rows[b_pre], acc_sh.at[dst_v.at[t - 2]],
                        ssem.at[b_pre]).wait()

                @pl.when(t + 2 < nch)
                def _():
                    pltpu.async_copy(hs.at[src_v.at[t + 2]], rows[b_pre],
                                     gsem.at[b_pre])

                @pl.when(t < nch)
                def _():
                    pltpu.make_async_copy(hs.at[src_v.at[t]], rows[b_cur],
                                          gsem.at[b_cur]).wait()
                    pltpu.async_copy(rows[b_cur], acc_sh.at[dst_v.at[t]],
                                     ssem.at[b_cur], add=True)

            return carry

        lax.fori_loop(0, (nch + 2 + 3) // 4, quad, 0)
        plsc.subcore_barrier()

        @pl.when(s < NS - 1)
        def _():
            pltpu.sync_copy(acc_sh.at[pl.ds(off, base)],
                            out_hbm.at[half, c, pl.ds(off, base)])

        @pl.when(s == NS - 1)
        def _():
            pltpu.sync_copy(acc_sh.at[pl.ds(lo_last, last)],
                            out_hbm.at[half, c, pl.ds(lo_last, last)])


def _make_sc_kernels(n, nch, ch):
    n1 = (n + 127) // 128 * 128
    deg_k = pl.kernel(
        _deg_body,
        out_type=jax.ShapeDtypeStruct((NC * n1,), jnp.float32),
        mesh=_MESH,
        scratch_types=[
            pltpu.VMEM((nch, ch), jnp.int32),
            pltpu.VMEM((ch,), jnp.float32),
            pltpu.VMEM_SHARED((n1,), jnp.float32),
            pltpu.SemaphoreType.DMA,
        ],
    )
    agg_k = pl.kernel(
        _agg_body,
        out_type=jax.ShapeDtypeStruct((2, NC, n, DH), jnp.float32),
        mesh=_MESH,
        compiler_params=pltpu.CompilerParams(use_tc_tiling_on_sc=False),
        scratch_types=[
            pltpu.VMEM((nch, ch), jnp.int32),
            pltpu.VMEM((nch, ch), jnp.int32),
            pltpu.VMEM((ch, DH), jnp.float32),
            pltpu.VMEM((ch, DH), jnp.float32),
            pltpu.VMEM((ch, DH), jnp.float32),
            pltpu.VMEM((ch, DH), jnp.float32),
            pltpu.SemaphoreType.DMA((4,)),
            pltpu.SemaphoreType.DMA((4,)),
            pltpu.VMEM_SHARED((n, DH), jnp.float32),
        ],
    )
    return deg_k, agg_k


def _tc1_body(x_ref, w1_ref, degp_ref, h1s0_ref, h1s1_ref, dis_ref):
    deg = degp_ref[0] + degp_ref[1] + 1.0
    dis = lax.rsqrt(deg)
    h = jnp.dot(x_ref[...], w1_ref[...], preferred_element_type=jnp.float32)
    hs = h * dis
    h1s0_ref[...] = hs[:, :DH]
    h1s1_ref[...] = hs[:, DH:]
    dis_ref[...] = dis


def _tc2_body(agg_ref, h1s0_ref, h1s1_ref, dis_ref, b1_ref, w2_ref,
              h2s0_ref, h2s1_ref):
    dis = dis_ref[...]
    b1 = b1_ref[...]
    pre0 = (agg_ref[0, 0] + agg_ref[0, 1] + h1s0_ref[...]) * dis + b1[:, :DH]
    pre1 = (agg_ref[1, 0] + agg_ref[1, 1] + h1s1_ref[...]) * dis + b1[:, DH:]
    h1 = jnp.concatenate([jnp.maximum(pre0, 0.0), jnp.maximum(pre1, 0.0)],
                         axis=1)
    h2 = jnp.dot(h1, w2_ref[...], preferred_element_type=jnp.float32)
    hs = h2 * dis
    h2s0_ref[...] = hs[:, :DH]
    h2s1_ref[...] = hs[:, DH:]


def _tc3_body(agg_ref, h2s0_ref, h2s1_ref, dis_ref, b2_ref, out_ref):
    dis = dis_ref[...]
    b2 = b2_ref[...]
    o0 = (agg_ref[0, 0] + agg_ref[0, 1] + h2s0_ref[...]) * dis + b2[:, :DH]
    o1 = (agg_ref[1, 0] + agg_ref[1, 1] + h2s1_ref[...]) * dis + b2[:, DH:]
    out_ref[...] = jnp.concatenate([o0, o1], axis=1)


def kernel(x, edge_index, W1, b1, W2, b2):
    n, _ = x.shape
    d_hid = W1.shape[1]
    d_out = W2.shape[1]
    e = edge_index.shape[1]
    assert d_hid == 2 * DH and d_out == 2 * DH

    per_w = e // NW
    ch = 80
    while per_w % ch:
        ch -= 8
    nch = per_w // ch

    ei = edge_index.astype(jnp.int32)
    src = ei[0].reshape(NW, nch, ch)
    dst = ei[1].reshape(NW, nch, ch)
    n1 = (n + 127) // 128 * 128
    zeros1 = jnp.zeros((n1,), jnp.float32)
    zeros2 = jnp.zeros((n, DH), jnp.float32)

    deg_k, agg_k = _make_sc_kernels(n, nch, ch)

    degp = deg_k(dst, zeros1)
    degp3 = degp.reshape(NC, n1, 1)[:, :n, :]

    tc1 = pl.pallas_call(
        _tc1_body,
        out_shape=(
            jax.ShapeDtypeStruct((n, DH), jnp.float32),
            jax.ShapeDtypeStruct((n, DH), jnp.float32),
            jax.ShapeDtypeStruct((n, 1), jnp.float32),
        ),
    )
    h1s0, h1s1, dis = tc1(x, W1, degp3)

    agg1 = agg_k(h1s0, h1s1, src, dst, zeros2)

    tc2 = pl.pallas_call(
        _tc2_body,
        out_shape=(
            jax.ShapeDtypeStruct((n, DH), jnp.float32),
            jax.ShapeDtypeStruct((n, DH), jnp.float32),
        ),
    )
    h2s0, h2s1 = tc2(agg1, h1s0, h1s1, dis, b1.reshape(1, d_hid), W2)

    agg2 = agg_k(h2s0, h2s1, src, dst, zeros2)

    tc3 = pl.pallas_call(
        _tc3_body,
        out_shape=jax.ShapeDtypeStruct((n, d_out), jnp.float32),
    )
    out = tc3(agg2, h2s0, h2s1, dis, b2.reshape(1, d_out))
    return out
